# Initial kernel scaffold; baseline (speedup 1.0000x reference)
import jax
import jax.numpy as jnp
from jax.experimental import pallas as pl


def kernel(x, W1, b1, W2, b2):
    raise NotImplementedError("write your pallas kernel here")


# speedup vs baseline: 1.7127x; 1.7127x over previous
import functools

import jax
import jax.numpy as jnp
from jax.experimental import pallas as pl
from jax.experimental.pallas import tpu as pltpu

HIDDEN = 4096
NUM_EXPERTS = 64
TOP_K = 8

BT = 1024
BH = 512

_INTERPRET = False


def _router_kernel(x_ref, w1_ref, b1_ref, w2_ref, b2_ref,
                   rw_ref, idx_ref, tkw_ref, acc_ref, *, n_h_blocks):
    j = pl.program_id(1)

    h = jax.lax.dot_general(
        x_ref[...], w1_ref[...],
        dimension_numbers=(((1,), (1,)), ((), ())),
        preferred_element_type=jnp.float32)
    h = jnp.maximum(h + b1_ref[0, :], 0.0)

    part = jax.lax.dot_general(
        h, w2_ref[...],
        dimension_numbers=(((1,), (1,)), ((), ())),
        preferred_element_type=jnp.float32)

    @pl.when(j == 0)
    def _():
        acc_ref[...] = part

    @pl.when(j > 0)
    def _():
        acc_ref[...] += part

    @pl.when(j == n_h_blocks - 1)
    def _():
        logits = acc_ref[...] + b2_ref[0, :]
        m = jnp.max(logits, axis=-1, keepdims=True)
        e = jnp.exp(logits - m)
        w = e / jnp.sum(e, axis=-1, keepdims=True)
        rw_ref[...] = w

        lane = jax.lax.broadcasted_iota(jnp.int32, (BT, NUM_EXPERTS), 1)
        vals = w
        idx_cols = []
        val_cols = []
        for _ in range(TOP_K):
            mx = jnp.max(vals, axis=-1, keepdims=True)
            amx = jnp.min(jnp.where(vals == mx, lane, NUM_EXPERTS),
                          axis=-1, keepdims=True)
            idx_cols.append(amx)
            val_cols.append(mx)
            vals = jnp.where(lane == amx, -jnp.inf, vals)
        idx8 = jnp.concatenate(idx_cols, axis=1)
        w8 = jnp.concatenate(val_cols, axis=1)
        idx_ref[...] = idx8
        tkw_ref[...] = w8 / jnp.sum(w8, axis=-1, keepdims=True)


def kernel(x, W1, b1, W2, b2):
    B, T, K = x.shape
    n_tok = B * T
    x2 = x.reshape(n_tok, K)
    b1r = b1.reshape(1, K)
    b2r = b2.reshape(1, NUM_EXPERTS)

    n_i = n_tok // BT
    n_j = K // BH

    out_shapes = (
        jax.ShapeDtypeStruct((n_tok, NUM_EXPERTS), jnp.float32),
        jax.ShapeDtypeStruct((n_tok, TOP_K), jnp.int32),
        jax.ShapeDtypeStruct((n_tok, TOP_K), jnp.float32),
    )

    rw, idx, tkw = pl.pallas_call(
        functools.partial(_router_kernel, n_h_blocks=n_j),
        grid=(n_i, n_j),
        in_specs=[
            pl.BlockSpec((BT, K), lambda i, j: (i, 0)),
            pl.BlockSpec((BH, K), lambda i, j: (j, 0)),
            pl.BlockSpec((1, BH), lambda i, j: (0, j)),
            pl.BlockSpec((NUM_EXPERTS, BH), lambda i, j: (0, j)),
            pl.BlockSpec((1, NUM_EXPERTS), lambda i, j: (0, 0)),
        ],
        out_specs=[
            pl.BlockSpec((BT, NUM_EXPERTS), lambda i, j: (i, 0)),
            pl.BlockSpec((BT, TOP_K), lambda i, j: (i, 0)),
            pl.BlockSpec((BT, TOP_K), lambda i, j: (i, 0)),
        ],
        out_shape=out_shapes,
        scratch_shapes=[pltpu.VMEM((BT, NUM_EXPERTS), jnp.float32)],
        compiler_params=pltpu.CompilerParams(
            dimension_semantics=("parallel", "arbitrary")),
        interpret=_INTERPRET,
    )(x2, W1, b1r, W2, b2r)

    return (rw.reshape(B, T, NUM_EXPERTS),
            idx.reshape(B, T, TOP_K),
            tkw.reshape(B, T, TOP_K))


# speedup vs baseline: 1.7746x; 1.0362x over previous
import functools

import jax
import jax.numpy as jnp
from jax.experimental import pallas as pl
from jax.experimental.pallas import tpu as pltpu

HIDDEN = 4096
NUM_EXPERTS = 64
TOP_K = 8

BT = 1024
BH = 512

_INTERPRET = False


def _router_kernel(x_ref, w1_ref, b1_ref, w2_ref, b2_ref,
                   rw_ref, idx_ref, tkw_ref, acc_ref, *, n_h_blocks):
    j = pl.program_id(1)

    h = jax.lax.dot_general(
        x_ref[...], w1_ref[...],
        dimension_numbers=(((1,), (1,)), ((), ())),
        preferred_element_type=jnp.float32)
    h = jnp.maximum(h + b1_ref[0, :], 0.0)

    part = jax.lax.dot_general(
        w2_ref[...], h,
        dimension_numbers=(((1,), (1,)), ((), ())),
        preferred_element_type=jnp.float32)

    @pl.when(j == 0)
    def _():
        acc_ref[...] = part

    @pl.when(j > 0)
    def _():
        acc_ref[...] += part

    @pl.when(j == n_h_blocks - 1)
    def _():
        logits = acc_ref[...] + b2_ref[:, 0:1]
        m = jnp.max(logits, axis=0, keepdims=True)
        e = jnp.exp(logits - m)
        w = e * (1.0 / jnp.sum(e, axis=0, keepdims=True))
        rw_ref[...] = w.T

        expert = jax.lax.broadcasted_iota(jnp.int32, (NUM_EXPERTS, BT), 0)
        vals = w
        idx_rows = []
        val_rows = []
        for _ in range(TOP_K):
            mx = jnp.max(vals, axis=0, keepdims=True)
            amx = jnp.min(jnp.where(vals == mx, expert, NUM_EXPERTS),
                          axis=0, keepdims=True)
            idx_rows.append(amx)
            val_rows.append(mx)
            vals = jnp.where(expert == amx, -jnp.inf, vals)
        idx8 = jnp.concatenate(idx_rows, axis=0)
        w8 = jnp.concatenate(val_rows, axis=0)
        idx_ref[...] = idx8.T
        tkw_ref[...] = (w8 * (1.0 / jnp.sum(w8, axis=0, keepdims=True))).T


def kernel(x, W1, b1, W2, b2):
    B, T, K = x.shape
    n_tok = B * T
    x2 = x.reshape(n_tok, K)
    b1r = b1.reshape(1, K)
    b2r = b2.reshape(NUM_EXPERTS, 1)

    n_i = n_tok // BT
    n_j = K // BH

    out_shapes = (
        jax.ShapeDtypeStruct((n_tok, NUM_EXPERTS), jnp.float32),
        jax.ShapeDtypeStruct((n_tok, TOP_K), jnp.int32),
        jax.ShapeDtypeStruct((n_tok, TOP_K), jnp.float32),
    )

    rw, idx, tkw = pl.pallas_call(
        functools.partial(_router_kernel, n_h_blocks=n_j),
        grid=(n_i, n_j),
        in_specs=[
            pl.BlockSpec((BT, K), lambda i, j: (i, 0)),
            pl.BlockSpec((BH, K), lambda i, j: (j, 0)),
            pl.BlockSpec((1, BH), lambda i, j: (0, j)),
            pl.BlockSpec((NUM_EXPERTS, BH), lambda i, j: (0, j)),
            pl.BlockSpec((NUM_EXPERTS, 1), lambda i, j: (0, 0)),
        ],
        out_specs=[
            pl.BlockSpec((BT, NUM_EXPERTS), lambda i, j: (i, 0)),
            pl.BlockSpec((BT, TOP_K), lambda i, j: (i, 0)),
            pl.BlockSpec((BT, TOP_K), lambda i, j: (i, 0)),
        ],
        out_shape=out_shapes,
        scratch_shapes=[pltpu.VMEM((NUM_EXPERTS, BT), jnp.float32)],
        compiler_params=pltpu.CompilerParams(
            dimension_semantics=("parallel", "arbitrary")),
        interpret=_INTERPRET,
    )(x2, W1, b1r, W2, b2r)

    return (rw.reshape(B, T, NUM_EXPERTS),
            idx.reshape(B, T, TOP_K),
            tkw.reshape(B, T, TOP_K))
